# baseline (speedup 1.0000x reference)
import jax
import jax.numpy as jnp
from jax.experimental import pallas as pl

_ALPHA = 0.1
_N = 8
_BLK = 128


def _dotT(W, v):
    return jax.lax.dot_general(
        W, v, dimension_numbers=(((0,), (0,)), ((), ())),
        preferred_element_type=jnp.float32)


def _gat(feat, maskf, ea, Wl, blc, Wr, brc, Wec, attc, biasc):
    fi, n, blk = feat.shape
    fo = Wl.shape[1]
    f2 = feat.reshape(fi, n * blk)
    xl = (_dotT(Wl, f2) + blc).reshape(fo, n, blk)
    xr = (_dotT(Wr, f2) + brc).reshape(fo, n, blk)
    m = (xr[:, :, None, :] + xl[:, None, :, :]
         + ea[None, :, :, :] * Wec.reshape(fo, 1, 1, 1))
    s = jnp.where(m >= 0, m, 0.2 * m)
    logits = _dotT(attc, s.reshape(fo, n * n * blk)).reshape(n, n, blk)
    neg = jnp.where(maskf > 0, logits, -1e30)
    mx = jnp.max(neg, axis=1, keepdims=True)
    ex = jnp.where(maskf > 0, jnp.exp(neg - mx), 0.0)
    denom = jnp.maximum(jnp.sum(ex, axis=1, keepdims=True), 1e-16)
    alpha = ex / denom
    acc = biasc.reshape(fo, 1, 1) + jnp.zeros((fo, n, blk), jnp.float32)
    for j in range(n):
        acc = acc + alpha[:, j, :][None, :, :] * xl[:, j, :][:, None, :]
    return acc


def _fwd(x_ref, W1, b1, W2, b2, W3, b3,
         Wl1, bl1, Wr1, br1, We1, att1, bias1,
         Wl2, bl2, Wr2, br2, We2, att2, bias2,
         Wl3, bl3, Wr3, br3, We3, att3, bias3,
         Wskip, bskip, rec_ref, lat_ref, mask_ref):
    blk = x_ref.shape[1]
    xb = x_ref[...]
    idx = jax.lax.broadcasted_iota(
        jnp.int32, (1, _N, blk), 1).astype(jnp.float32)
    pre = jnp.concatenate(
        [jnp.zeros((1, _N, blk), jnp.float32), idx, xb[None]], axis=0)
    pre2 = pre.reshape(3, _N * blk)
    h = jnp.maximum(_dotT(W1[...], pre2) + b1[...], 0.0)
    h = jnp.maximum(_dotT(W2[...], h) + b2[...], 0.0)
    lat2 = _dotT(W3[...], h) + b3[...]
    lat = lat2.reshape(3, _N, blk)
    lat_ref[...] = lat

    p = lat
    diff = p[:, :, None, :] - p[:, None, :, :]
    dist2 = jnp.sum(diff * diff, axis=0)
    ea = jnp.sqrt(dist2 + 1e-12)
    mid = (p[:, :, None, :] + p[:, None, :, :]) * 0.5
    r2 = jnp.zeros((_N, _N, blk), jnp.float32)
    d2 = jnp.zeros((_N, _N, _N, blk), jnp.float32)
    for c in range(3):
        dm = p[c][:, None, :] - mid[c]
        r2 = r2 + dm * dm
        dd = p[c][None, None, :, :] - mid[c][:, :, None, :]
        d2 = d2 + dd * dd
    ii = jax.lax.broadcasted_iota(jnp.int32, (_N, _N, _N, 1), 0)
    jj = jax.lax.broadcasted_iota(jnp.int32, (_N, _N, _N, 1), 1)
    kk = jax.lax.broadcasted_iota(jnp.int32, (_N, _N, _N, 1), 2)
    excl = (kk == ii) | (kk == jj)
    violf = jnp.max(
        jnp.where(excl | (d2 >= r2[:, :, None, :]), 0.0, 1.0),
        axis=2)
    i2 = jax.lax.broadcasted_iota(jnp.int32, (_N, _N, 1), 0)
    j2 = jax.lax.broadcasted_iota(jnp.int32, (_N, _N, 1), 1)
    maskf = jnp.where((violf == 0) & (i2 != j2), 1.0, 0.0)
    mask_ref[...] = maskf

    feat = lat[2:3]
    x1 = jnp.maximum(
        _gat(feat, maskf, ea, Wl1[...], bl1[...], Wr1[...], br1[...],
             We1[...], att1[...], bias1[...]), 0.0)
    x2 = jnp.maximum(
        _gat(x1, maskf, ea, Wl2[...], bl2[...], Wr2[...], br2[...],
             We2[...], att2[...], bias2[...]), 0.0)
    g3 = _gat(x2, maskf, ea, Wl3[...], bl3[...], Wr3[...], br3[...],
              We3[...], att3[...], bias3[...])
    skip = _dotT(Wskip[...], lat2) + bskip[...]
    rec_ref[...] = g3 + _ALPHA * skip.reshape(3, _N, blk)


def kernel(x, W1, b1, W2, b2, W3, b3,
           Wl1, bl1, Wr1, br1, We1, att1, bias1,
           Wl2, bl2, Wr2, br2, We2, att2, bias2,
           Wl3, bl3, Wr3, br3, We3, att3, bias3,
           Wskip, bskip):
    B = x.shape[0]
    f32 = jnp.float32
    col = lambda v: v.reshape(-1, 1)
    args = (x.T, W1, col(b1), W2, col(b2), W3, col(b3),
            Wl1, col(bl1), Wr1, col(br1), col(We1), col(att1), col(bias1),
            Wl2, col(bl2), Wr2, col(br2), col(We2), col(att2), col(bias2),
            Wl3, col(bl3), Wr3, col(br3), col(We3), col(att3), col(bias3),
            Wskip, col(bskip))
    in_specs = [pl.BlockSpec((_N, _BLK), lambda j: (0, j))]
    in_specs += [pl.BlockSpec(a.shape, lambda j: (0, 0)) for a in args[1:]]
    out_specs = [pl.BlockSpec((3, _N, _BLK), lambda j: (0, 0, j)),
                 pl.BlockSpec((3, _N, _BLK), lambda j: (0, 0, j)),
                 pl.BlockSpec((_N, _N, _BLK), lambda j: (0, 0, j))]
    out_shape = [jax.ShapeDtypeStruct((3, _N, B), f32),
                 jax.ShapeDtypeStruct((3, _N, B), f32),
                 jax.ShapeDtypeStruct((_N, _N, B), f32)]
    rec_t, lat_t, mask_t = pl.pallas_call(
        _fwd, grid=(B // _BLK,),
        in_specs=in_specs, out_specs=out_specs, out_shape=out_shape)(*args)
    idxf = jnp.arange(_N, dtype=f32)
    pre = jnp.stack(
        [jnp.zeros_like(x), jnp.broadcast_to(idxf, x.shape), x], axis=-1)
    rec = rec_t.transpose(2, 1, 0)
    latent = lat_t.transpose(2, 1, 0)
    mask = mask_t.transpose(2, 0, 1) > 0.5
    return pre, rec, latent, mask
